# baseline (speedup 1.0000x reference)
import functools
import math

import jax
import jax.numpy as jnp
from jax import lax
from jax.experimental import pallas as pl
from jax.experimental.pallas import tpu as pltpu
from jax.experimental.pallas import tpu_sc as plsc


_KB = 128


def _s_body(q_ref, k_ref, s_ref):
    q = q_ref[0, 0]
    kk = k_ref[0, 0]
    L = q.shape[0]
    for j in range(L // (2 * _KB)):
        kj = kk[j * 2 * _KB:(j + 1) * 2 * _KB, :]
        s = lax.dot_general(
            q, kj, (((1,), (1,)), ((), ())),
            preferred_element_type=jnp.float32)
        s_ref[0, 2 * j] = s[:, :_KB]
        s_ref[0, 2 * j + 1] = s[:, _KB:]


def _compute_scores(q4, k4, h0, nh):
    _, H, L, D = q4.shape
    return pl.pallas_call(
        _s_body,
        grid=(nh,),
        in_specs=[
            pl.BlockSpec((1, 1, L, D), lambda h: (0, h0 + h, 0, 0)),
            pl.BlockSpec((1, 1, L, D), lambda h: (0, h0 + h, 0, 0)),
        ],
        out_specs=pl.BlockSpec((1, L // _KB, L, _KB), lambda h: (h, 0, 0, 0)),
        out_shape=jax.ShapeDtypeStruct((nh, L // _KB, L, _KB), jnp.float32),
    )(q4, k4)


_W = 128
_NWIN = 320


def _sc_gather_m(table, addr, n_rows, u, l_k):
    n_sub, n_win, w = addr.shape
    rows_per = n_rows // n_sub
    n_grp = rows_per // 16
    assert n_grp * u * 16 == n_win * w
    mesh = plsc.VectorSubcoreMesh(
        core_axis_name="core", subcore_axis_name="subcore")

    @functools.partial(
        pl.kernel, mesh=mesh,
        out_type=jax.ShapeDtypeStruct((n_rows,), jnp.float32),
        scratch_types=[
            pltpu.VMEM((n_win, w), jnp.int32),
            pltpu.VMEM((n_win, w), jnp.float32),
            pltpu.VMEM((rows_per,), jnp.float32),
            pltpu.SemaphoreType.DMA,
            pltpu.SemaphoreType.DMA,
        ])
    def gather_kernel(x_hbm, i_hbm, o_hbm, idx_v, val_v, m_v, sem_i, sem_g):
        wid = lax.axis_index("core") * 16 + lax.axis_index("subcore")
        pltpu.async_copy(i_hbm.at[wid], idx_v, sem_i).wait()

        @pl.loop(0, n_win)
        def _fire(win):
            pltpu.async_copy(x_hbm.at[idx_v.at[win]], val_v.at[win], sem_g)

        @pl.loop(0, n_win)
        def _drain(win):
            pltpu.make_async_copy(
                x_hbm.at[idx_v.at[win]], val_v.at[win], sem_g).wait()

        for g in range(n_grp):
            base = g * 16 * u
            r0, c0 = base // w, base % w
            mx = val_v[r0, pl.ds(c0, 16)]
            sm = mx
            for s in range(1, u):
                off = base + s * 16
                v = val_v[off // w, pl.ds(off % w, 16)]
                mx = jnp.maximum(mx, v)
                sm = sm + v
            m_v[pl.ds(g * 16, 16)] = mx - sm * (1.0 / l_k)

        pltpu.sync_copy(m_v, o_hbm.at[pl.ds(wid * rows_per, rows_per)])

    return gather_kernel(table, addr)


def _topk_body(m_ref, idx_ref, *, u):
    m = m_ref[...]
    H, LB, C = m.shape
    gidx = (lax.broadcasted_iota(jnp.int32, (H, LB, C), 1) * C
            + lax.broadcasted_iota(jnp.int32, (H, LB, C), 2))
    neg = jnp.float32(-3.0e38)
    big = jnp.int32(2**30)
    for i in range(u):
        rm = jnp.max(jnp.max(m, axis=2, keepdims=True), axis=1,
                     keepdims=True)
        cand = jnp.where(m >= rm, gidx, big)
        pos = jnp.min(jnp.min(cand, axis=2, keepdims=True), axis=1,
                      keepdims=True)
        idx_ref[:, :, pl.ds(i, 1)] = pos
        m = jnp.where(gidx == pos, neg, m)


def _topk(m3, u):
    H, LB, C = m3.shape
    return pl.pallas_call(
        functools.partial(_topk_body, u=u),
        grid=(1,),
        in_specs=[pl.BlockSpec((H, LB, C), lambda i: (0, 0, 0))],
        out_specs=pl.BlockSpec((H, 1, u), lambda i: (0, 0, 0)),
        out_shape=jax.ShapeDtypeStruct((H, 1, u), jnp.int32),
    )(m3)


_CB = 128


def _ctx_body(mtc_ref, mtr_ref, q_ref, k_ref, v_ref, o_ref, *, scale):
    mt_col = mtc_ref[0]
    mt_row = mtr_ref[0]
    q = q_ref[0, 0]
    k = k_ref[0, 0]
    v = v_ref[0, 0]
    L, D = q.shape
    U = mt_col.shape[0]
    f32 = jnp.float32

    oh_ul = (lax.broadcasted_iota(jnp.int32, (U, L), 1) == mt_col)
    oh_lu = (lax.broadcasted_iota(jnp.int32, (L, U), 0) == mt_row)

    qr = lax.dot_general(
        oh_ul.astype(f32), q, (((1,), (0,)), ((), ())),
        preferred_element_type=f32)

    st = lax.dot_general(
        qr, k, (((1,), (1,)), ((), ())),
        preferred_element_type=f32) * f32(scale)

    key_iota = lax.broadcasted_iota(jnp.int32, (U, L), 1)
    st = jnp.where(key_iota > mt_col, -jnp.inf, st)

    smax = jnp.max(st, axis=1, keepdims=True)
    e = jnp.exp(st - smax)
    attn = e / jnp.sum(e, axis=1, keepdims=True)

    upd = lax.dot_general(
        attn, v, (((1,), (0,)), ((), ())),
        preferred_element_type=f32)

    scat = lax.dot_general(
        oh_lu.astype(f32), upd, (((1,), (0,)), ((), ())),
        preferred_element_type=f32)

    selrow = jnp.sum(oh_lu.astype(f32), axis=1, keepdims=True) > 0

    tri = (lax.broadcasted_iota(jnp.int32, (_CB, _CB), 0)
           >= lax.broadcasted_iota(jnp.int32, (_CB, _CB), 1)).astype(f32)
    carry = jnp.zeros((1, D), f32)
    for b in range(L // _CB):
        lo = b * _CB
        blk = v[lo:lo + _CB, :]
        c = lax.dot_general(
            tri, blk, (((1,), (0,)), ((), ())),
            preferred_element_type=f32,
            precision=lax.Precision.HIGHEST) + carry
        o_ref[0, 0, lo:lo + _CB, :] = jnp.where(
            selrow[lo:lo + _CB, :], scat[lo:lo + _CB, :], c)
        carry = carry + jnp.sum(blk, axis=0, keepdims=True)


def _ctx(mt_col3, mt_row3, q4, k4, v4, scale, h0, nh):
    _, H, L, D = q4.shape
    U = mt_col3.shape[1]
    spec_hld = pl.BlockSpec((1, 1, L, D), lambda h: (0, h0 + h, 0, 0))
    return pl.pallas_call(
        functools.partial(_ctx_body, scale=scale),
        grid=(nh,),
        in_specs=[
            pl.BlockSpec((1, U, 1), lambda h: (h, 0, 0)),
            pl.BlockSpec((1, 1, U), lambda h: (h, 0, 0)),
            spec_hld, spec_hld, spec_hld,
        ],
        out_specs=pl.BlockSpec((1, 1, L, D), lambda h: (0, h, 0, 0)),
        out_shape=jax.ShapeDtypeStruct((1, nh, L, D), jnp.float32),
    )(mt_col3, mt_row3, q4, k4, v4)


_NCHUNK = 4


def kernel(queries, keys, values, attn_mask):
    B, H, L_Q, E = queries.shape
    L_K = keys.shape[2]
    factor = 5
    scale = 1.0 / math.sqrt(E)
    u_part = min(factor * math.ceil(math.log(L_K)), L_K)
    u = min(factor * math.ceil(math.log(L_Q)), L_Q)
    hc = H // _NCHUNK

    with jax.ensure_compile_time_eval():
        skey = jax.random.key(12345)
        idx = jax.random.randint(skey, (L_Q, u_part), 0, L_K)
        hb = jnp.arange(hc, dtype=jnp.int32)[:, None, None]
        lb = jnp.arange(L_Q, dtype=jnp.int32)[None, :, None]
        jb = (idx // _KB).astype(jnp.int32)[None]
        cb = (idx % _KB).astype(jnp.int32)[None]
        addr_c = ((hb * (L_K // _KB) + jb) * (L_Q * _KB)
                  + lb * _KB + cb)
        addr_c = (addr_c.reshape(32, hc * L_Q // (32 * 16), 16, u_part)
                  .transpose(0, 1, 3, 2)
                  .reshape(32, hc * u_part * L_Q // (32 * _W), _W))

    out_parts = []
    for g in range(_NCHUNK):
        s_g = _compute_scores(queries, keys, g * hc, hc)
        m_g = _sc_gather_m(s_g.reshape(hc * L_Q * L_K), addr_c,
                           hc * L_Q, u_part, L_K)
        m3_g = m_g.reshape(hc, L_Q // 128, 128)
        mtop_g = _topk(m3_g, u)
        out_parts.append(_ctx(mtop_g.reshape(hc, u, 1),
                              mtop_g.reshape(hc, 1, u),
                              queries, keys, values, scale, g * hc, hc))
    return jnp.concatenate(out_parts, axis=1)


# speedup vs baseline: 1.1379x; 1.1379x over previous
import functools
import math

import jax
import jax.numpy as jnp
from jax import lax
from jax.experimental import pallas as pl
from jax.experimental.pallas import tpu as pltpu
from jax.experimental.pallas import tpu_sc as plsc


_KB = 128


def _s_body(q_ref, k_ref, s_ref):
    q = q_ref[0, 0]
    kk = k_ref[0, 0]
    L = q.shape[0]
    for j in range(L // (2 * _KB)):
        kj = kk[j * 2 * _KB:(j + 1) * 2 * _KB, :]
        s = lax.dot_general(
            q, kj, (((1,), (1,)), ((), ())),
            preferred_element_type=jnp.float32)
        s_ref[0, 2 * j] = s[:, :_KB]
        s_ref[0, 2 * j + 1] = s[:, _KB:]


def _compute_scores(q4, k4, h0, nh):
    _, H, L, D = q4.shape
    return pl.pallas_call(
        _s_body,
        grid=(nh,),
        in_specs=[
            pl.BlockSpec((1, 1, L, D), lambda h: (0, h0 + h, 0, 0)),
            pl.BlockSpec((1, 1, L, D), lambda h: (0, h0 + h, 0, 0)),
        ],
        out_specs=pl.BlockSpec((1, L // _KB, L, _KB), lambda h: (h, 0, 0, 0)),
        out_shape=jax.ShapeDtypeStruct((nh, L // _KB, L, _KB), jnp.float32),
    )(q4, k4)


_W = 128
_NWIN = 320


def _sc_gather_m(table, addr, n_rows, u, l_k):
    n_sub, n_win, w = addr.shape
    rows_per = n_rows // n_sub
    n_grp = rows_per // 16
    assert n_grp * u * 16 == n_win * w
    mesh = plsc.VectorSubcoreMesh(
        core_axis_name="core", subcore_axis_name="subcore")

    @functools.partial(
        pl.kernel, mesh=mesh,
        out_type=jax.ShapeDtypeStruct((n_rows,), jnp.float32),
        scratch_types=[
            pltpu.VMEM((n_win, w), jnp.int32),
            pltpu.VMEM((n_win, w), jnp.float32),
            pltpu.VMEM((rows_per,), jnp.float32),
            pltpu.SemaphoreType.DMA,
            pltpu.SemaphoreType.DMA,
        ])
    def gather_kernel(x_hbm, i_hbm, o_hbm, idx_v, val_v, m_v, sem_i, sem_g):
        wid = lax.axis_index("core") * 16 + lax.axis_index("subcore")
        pltpu.async_copy(i_hbm.at[wid], idx_v, sem_i).wait()

        @pl.loop(0, n_win)
        def _fire(win):
            pltpu.async_copy(x_hbm.at[idx_v.at[win]], val_v.at[win], sem_g)

        @pl.loop(0, n_win)
        def _drain(win):
            pltpu.make_async_copy(
                x_hbm.at[idx_v.at[win]], val_v.at[win], sem_g).wait()

        for g in range(n_grp):
            base = g * 16 * u
            r0, c0 = base // w, base % w
            mx = val_v[r0, pl.ds(c0, 16)]
            sm = mx
            for s in range(1, u):
                off = base + s * 16
                v = val_v[off // w, pl.ds(off % w, 16)]
                mx = jnp.maximum(mx, v)
                sm = sm + v
            m_v[pl.ds(g * 16, 16)] = mx - sm * (1.0 / l_k)

        pltpu.sync_copy(m_v, o_hbm.at[pl.ds(wid * rows_per, rows_per)])

    return gather_kernel(table, addr)


def _topk_body(m_ref, idx_ref, *, u):
    m = m_ref[...]
    H, LB, C = m.shape
    gidx = (lax.broadcasted_iota(jnp.int32, (H, LB, C), 1) * C
            + lax.broadcasted_iota(jnp.int32, (H, LB, C), 2))
    neg = jnp.float32(-3.0e38)
    big = jnp.int32(2**30)
    for i in range(u):
        rm = jnp.max(jnp.max(m, axis=2, keepdims=True), axis=1,
                     keepdims=True)
        cand = jnp.where(m >= rm, gidx, big)
        pos = jnp.min(jnp.min(cand, axis=2, keepdims=True), axis=1,
                      keepdims=True)
        idx_ref[:, :, pl.ds(i, 1)] = pos
        m = jnp.where(gidx == pos, neg, m)


def _topk(m3, u):
    H, LB, C = m3.shape
    return pl.pallas_call(
        functools.partial(_topk_body, u=u),
        grid=(1,),
        in_specs=[pl.BlockSpec((H, LB, C), lambda i: (0, 0, 0))],
        out_specs=pl.BlockSpec((H, 1, u), lambda i: (0, 0, 0)),
        out_shape=jax.ShapeDtypeStruct((H, 1, u), jnp.int32),
    )(m3)


_CB = 128


def _ctx_body(mtc_ref, mtr_ref, q_ref, k_ref, v_ref, o_ref, *, scale):
    mt_col = mtc_ref[0]
    mt_row = mtr_ref[0]
    q = q_ref[0, 0]
    k = k_ref[0, 0]
    v = v_ref[0, 0]
    L, D = q.shape
    U = mt_col.shape[0]
    f32 = jnp.float32

    oh_ul = (lax.broadcasted_iota(jnp.int32, (U, L), 1) == mt_col)
    oh_lu = (lax.broadcasted_iota(jnp.int32, (L, U), 0) == mt_row)

    qr = lax.dot_general(
        oh_ul.astype(f32), q, (((1,), (0,)), ((), ())),
        preferred_element_type=f32)

    st = lax.dot_general(
        qr, k, (((1,), (1,)), ((), ())),
        preferred_element_type=f32) * f32(scale)

    key_iota = lax.broadcasted_iota(jnp.int32, (U, L), 1)
    st = jnp.where(key_iota > mt_col, -jnp.inf, st)

    smax = jnp.max(st, axis=1, keepdims=True)
    e = jnp.exp(st - smax)
    attn = e / jnp.sum(e, axis=1, keepdims=True)

    upd = lax.dot_general(
        attn, v, (((1,), (0,)), ((), ())),
        preferred_element_type=f32)

    scat = lax.dot_general(
        oh_lu.astype(f32), upd, (((1,), (0,)), ((), ())),
        preferred_element_type=f32)

    selrow = jnp.sum(oh_lu.astype(f32), axis=1, keepdims=True) > 0

    tri = (lax.broadcasted_iota(jnp.int32, (_CB, _CB), 0)
           >= lax.broadcasted_iota(jnp.int32, (_CB, _CB), 1)).astype(f32)
    carry = jnp.zeros((1, D), f32)
    for b in range(L // _CB):
        lo = b * _CB
        blk = v[lo:lo + _CB, :]
        c = lax.dot_general(
            tri, blk, (((1,), (0,)), ((), ())),
            preferred_element_type=f32,
            precision=lax.Precision.HIGHEST) + carry
        o_ref[0, 0, lo:lo + _CB, :] = jnp.where(
            selrow[lo:lo + _CB, :], scat[lo:lo + _CB, :], c)
        carry = carry + jnp.sum(blk, axis=0, keepdims=True)


def _ctx(mt_col3, mt_row3, q4, k4, v4, scale, h0, nh):
    _, H, L, D = q4.shape
    U = mt_col3.shape[1]
    spec_hld = pl.BlockSpec((1, 1, L, D), lambda h: (0, h0 + h, 0, 0))
    return pl.pallas_call(
        functools.partial(_ctx_body, scale=scale),
        grid=(nh,),
        in_specs=[
            pl.BlockSpec((1, U, 1), lambda h: (h, 0, 0)),
            pl.BlockSpec((1, 1, U), lambda h: (h, 0, 0)),
            spec_hld, spec_hld, spec_hld,
        ],
        out_specs=pl.BlockSpec((1, 1, L, D), lambda h: (0, h, 0, 0)),
        out_shape=jax.ShapeDtypeStruct((1, nh, L, D), jnp.float32),
    )(mt_col3, mt_row3, q4, k4, v4)


_NCHUNK = 4


def kernel(queries, keys, values, attn_mask):
    B, H, L_Q, E = queries.shape
    L_K = keys.shape[2]
    factor = 5
    scale = 1.0 / math.sqrt(E)
    u_part = min(factor * math.ceil(math.log(L_K)), L_K)
    u = min(factor * math.ceil(math.log(L_Q)), L_Q)
    hc = H // _NCHUNK

    with jax.ensure_compile_time_eval():
        skey = jax.random.key(12345)
        idx = jax.random.randint(skey, (L_Q, u_part), 0, L_K)
        hb = jnp.arange(hc, dtype=jnp.int32)[:, None, None]
        lb = jnp.arange(L_Q, dtype=jnp.int32)[None, :, None]
        jb = (idx // _KB).astype(jnp.int32)[None]
        cb = (idx % _KB).astype(jnp.int32)[None]
        addr_c = ((hb * (L_K // _KB) + jb) * (L_Q * _KB)
                  + lb * _KB + cb)
        addr_c = (addr_c.reshape(32, hc * L_Q // (32 * 16), 16, u_part)
                  .transpose(0, 1, 3, 2)
                  .reshape(32, hc * u_part * L_Q // (32 * _W), _W))

    m_parts = []
    for g in range(_NCHUNK):
        s_g = _compute_scores(queries, keys, g * hc, hc)
        m_g = _sc_gather_m(s_g.reshape(hc * L_Q * L_K), addr_c,
                           hc * L_Q, u_part, L_K)
        m_parts.append(m_g.reshape(hc, L_Q // 128, 128))

    m3 = jnp.concatenate(m_parts, axis=0)
    mtop = _topk(m3, u)
    return _ctx(mtop.reshape(H, u, 1), mtop.reshape(H, 1, u),
                queries, keys, values, scale, 0, H)


# speedup vs baseline: 1.1550x; 1.0150x over previous
import functools
import math

import jax
import jax.numpy as jnp
from jax import lax
from jax.experimental import pallas as pl
from jax.experimental.pallas import tpu as pltpu
from jax.experimental.pallas import tpu_sc as plsc


_KB = 128


def _s_body(q_ref, k_ref, s_ref):
    q = q_ref[0, 0]
    kk = k_ref[0, 0]
    L = q.shape[0]
    for j in range(L // (2 * _KB)):
        kj = kk[j * 2 * _KB:(j + 1) * 2 * _KB, :]
        s = lax.dot_general(
            q, kj, (((1,), (1,)), ((), ())),
            preferred_element_type=jnp.float32)
        s_ref[0, 2 * j] = s[:, :_KB]
        s_ref[0, 2 * j + 1] = s[:, _KB:]


def _compute_scores(q4, k4, h0, nh):
    _, H, L, D = q4.shape
    return pl.pallas_call(
        _s_body,
        grid=(nh,),
        in_specs=[
            pl.BlockSpec((1, 1, L, D), lambda h: (0, h0 + h, 0, 0)),
            pl.BlockSpec((1, 1, L, D), lambda h: (0, h0 + h, 0, 0)),
        ],
        out_specs=pl.BlockSpec((1, L // _KB, L, _KB), lambda h: (h, 0, 0, 0)),
        out_shape=jax.ShapeDtypeStruct((nh, L // _KB, L, _KB), jnp.float32),
    )(q4, k4)


_W = 128
_NWIN = 320


def _sc_gather_m(table, addr, n_rows, u, l_k):
    n_sub, n_win, w = addr.shape
    rows_per = n_rows // n_sub
    n_grp = rows_per // 16
    assert n_grp * u * 16 == n_win * w
    mesh = plsc.VectorSubcoreMesh(
        core_axis_name="core", subcore_axis_name="subcore")

    @functools.partial(
        pl.kernel, mesh=mesh,
        out_type=jax.ShapeDtypeStruct((n_rows,), jnp.float32),
        scratch_types=[
            pltpu.VMEM((n_win, w), jnp.int32),
            pltpu.VMEM((n_win, w), jnp.float32),
            pltpu.VMEM((rows_per,), jnp.float32),
            pltpu.SemaphoreType.DMA,
            pltpu.SemaphoreType.DMA,
        ])
    def gather_kernel(x_hbm, i_hbm, o_hbm, idx_v, val_v, m_v, sem_i, sem_g):
        wid = lax.axis_index("core") * 16 + lax.axis_index("subcore")
        pltpu.async_copy(i_hbm.at[wid], idx_v, sem_i).wait()

        @pl.loop(0, n_win)
        def _fire(win):
            pltpu.async_copy(x_hbm.at[idx_v.at[win]], val_v.at[win], sem_g)

        @pl.loop(0, n_win)
        def _drain(win):
            pltpu.make_async_copy(
                x_hbm.at[idx_v.at[win]], val_v.at[win], sem_g).wait()

        for g in range(n_grp):
            base = g * 16 * u
            r0, c0 = base // w, base % w
            mx = val_v[r0, pl.ds(c0, 16)]
            sm = mx
            for s in range(1, u):
                off = base + s * 16
                v = val_v[off // w, pl.ds(off % w, 16)]
                mx = jnp.maximum(mx, v)
                sm = sm + v
            m_v[pl.ds(g * 16, 16)] = mx - sm * (1.0 / l_k)

        pltpu.sync_copy(m_v, o_hbm.at[pl.ds(wid * rows_per, rows_per)])

    return gather_kernel(table, addr)


def _topk_body(m_ref, idx_ref, *, u):
    m = m_ref[...]
    H, LB, C = m.shape
    gidx = (lax.broadcasted_iota(jnp.int32, (H, LB, C), 1) * C
            + lax.broadcasted_iota(jnp.int32, (H, LB, C), 2)
            ).astype(jnp.float32)
    neg = jnp.float32(-3.0e38)
    big = jnp.float32(2.0**30)
    for i in range(u):
        rm = jnp.max(jnp.max(m, axis=2, keepdims=True), axis=1,
                     keepdims=True)
        cand = jnp.where(m >= rm, gidx, big)
        pos = jnp.min(jnp.min(cand, axis=2, keepdims=True), axis=1,
                      keepdims=True)
        idx_ref[:, :, pl.ds(i, 1)] = pos.astype(jnp.int32)
        m = jnp.where(gidx == pos, neg, m)


def _topk(m3, u):
    H, LB, C = m3.shape
    return pl.pallas_call(
        functools.partial(_topk_body, u=u),
        grid=(1,),
        in_specs=[pl.BlockSpec((H, LB, C), lambda i: (0, 0, 0))],
        out_specs=pl.BlockSpec((H, 1, u), lambda i: (0, 0, 0)),
        out_shape=jax.ShapeDtypeStruct((H, 1, u), jnp.int32),
    )(m3)


_CB = 128


def _ctx_body(mtc_ref, mtr_ref, q_ref, k_ref, v_ref, o_ref, *, scale):
    mt_col = mtc_ref[0]
    mt_row = mtr_ref[0]
    q = q_ref[0, 0]
    k = k_ref[0, 0]
    v = v_ref[0, 0]
    L, D = q.shape
    U = mt_col.shape[0]
    f32 = jnp.float32

    oh_ul = (lax.broadcasted_iota(jnp.int32, (U, L), 1) == mt_col)
    oh_lu = (lax.broadcasted_iota(jnp.int32, (L, U), 0) == mt_row)

    qr = lax.dot_general(
        oh_ul.astype(f32), q, (((1,), (0,)), ((), ())),
        preferred_element_type=f32)

    st = lax.dot_general(
        qr, k, (((1,), (1,)), ((), ())),
        preferred_element_type=f32) * f32(scale)

    key_iota = lax.broadcasted_iota(jnp.int32, (U, L), 1)
    st = jnp.where(key_iota > mt_col, -jnp.inf, st)

    smax = jnp.max(st, axis=1, keepdims=True)
    e = jnp.exp(st - smax)
    attn = e / jnp.sum(e, axis=1, keepdims=True)

    upd = lax.dot_general(
        attn, v, (((1,), (0,)), ((), ())),
        preferred_element_type=f32)

    scat = lax.dot_general(
        oh_lu.astype(f32), upd, (((1,), (0,)), ((), ())),
        preferred_element_type=f32)

    selrow = jnp.sum(oh_lu.astype(f32), axis=1, keepdims=True) > 0

    tri = (lax.broadcasted_iota(jnp.int32, (_CB, _CB), 0)
           >= lax.broadcasted_iota(jnp.int32, (_CB, _CB), 1)).astype(f32)
    carry = jnp.zeros((1, D), f32)
    for b in range(L // _CB):
        lo = b * _CB
        blk = v[lo:lo + _CB, :]
        c = lax.dot_general(
            tri, blk, (((1,), (0,)), ((), ())),
            preferred_element_type=f32,
            precision=lax.Precision.HIGHEST) + carry
        o_ref[0, 0, lo:lo + _CB, :] = jnp.where(
            selrow[lo:lo + _CB, :], scat[lo:lo + _CB, :], c)
        carry = carry + jnp.sum(blk, axis=0, keepdims=True)


def _ctx(mt_col3, mt_row3, q4, k4, v4, scale, h0, nh):
    _, H, L, D = q4.shape
    U = mt_col3.shape[1]
    spec_hld = pl.BlockSpec((1, 1, L, D), lambda h: (0, h0 + h, 0, 0))
    return pl.pallas_call(
        functools.partial(_ctx_body, scale=scale),
        grid=(nh,),
        in_specs=[
            pl.BlockSpec((1, U, 1), lambda h: (h, 0, 0)),
            pl.BlockSpec((1, 1, U), lambda h: (h, 0, 0)),
            spec_hld, spec_hld, spec_hld,
        ],
        out_specs=pl.BlockSpec((1, 1, L, D), lambda h: (0, h, 0, 0)),
        out_shape=jax.ShapeDtypeStruct((1, nh, L, D), jnp.float32),
    )(mt_col3, mt_row3, q4, k4, v4)


_NCHUNK = 4


def kernel(queries, keys, values, attn_mask):
    B, H, L_Q, E = queries.shape
    L_K = keys.shape[2]
    factor = 5
    scale = 1.0 / math.sqrt(E)
    u_part = min(factor * math.ceil(math.log(L_K)), L_K)
    u = min(factor * math.ceil(math.log(L_Q)), L_Q)
    hc = H // _NCHUNK

    with jax.ensure_compile_time_eval():
        skey = jax.random.key(12345)
        idx = jax.random.randint(skey, (L_Q, u_part), 0, L_K)
        hb = jnp.arange(hc, dtype=jnp.int32)[:, None, None]
        lb = jnp.arange(L_Q, dtype=jnp.int32)[None, :, None]
        jb = (idx // _KB).astype(jnp.int32)[None]
        cb = (idx % _KB).astype(jnp.int32)[None]
        addr_c = ((hb * (L_K // _KB) + jb) * (L_Q * _KB)
                  + lb * _KB + cb)
        addr_c = (addr_c.reshape(32, hc * L_Q // (32 * 16), 16, u_part)
                  .transpose(0, 1, 3, 2)
                  .reshape(32, hc * u_part * L_Q // (32 * _W), _W))

    m_parts = []
    for g in range(_NCHUNK):
        s_g = _compute_scores(queries, keys, g * hc, hc)
        m_g = _sc_gather_m(s_g.reshape(hc * L_Q * L_K), addr_c,
                           hc * L_Q, u_part, L_K)
        m_parts.append(m_g.reshape(hc, L_Q // 128, 128))

    m3 = jnp.concatenate(m_parts, axis=0)
    mtop = _topk(m3, u)
    return _ctx(mtop.reshape(H, u, 1), mtop.reshape(H, 1, u),
                queries, keys, values, scale, 0, H)


# speedup vs baseline: 1.1600x; 1.0044x over previous
import functools
import math

import jax
import jax.numpy as jnp
from jax import lax
from jax.experimental import pallas as pl
from jax.experimental.pallas import tpu as pltpu
from jax.experimental.pallas import tpu_sc as plsc


_KB = 128


def _s_body(q_ref, k_ref, s_ref):
    q = q_ref[0, 0]
    kk = k_ref[0, 0]
    L = q.shape[0]
    for j in range(L // (2 * _KB)):
        kj = kk[j * 2 * _KB:(j + 1) * 2 * _KB, :]
        s = lax.dot_general(
            q, kj, (((1,), (1,)), ((), ())),
            preferred_element_type=jnp.float32)
        s_ref[0, 2 * j] = s[:, :_KB]
        s_ref[0, 2 * j + 1] = s[:, _KB:]


def _compute_scores(q4, k4, h0, nh):
    _, H, L, D = q4.shape
    return pl.pallas_call(
        _s_body,
        grid=(nh,),
        in_specs=[
            pl.BlockSpec((1, 1, L, D), lambda h: (0, h0 + h, 0, 0)),
            pl.BlockSpec((1, 1, L, D), lambda h: (0, h0 + h, 0, 0)),
        ],
        out_specs=pl.BlockSpec((1, L // _KB, L, _KB), lambda h: (h, 0, 0, 0)),
        out_shape=jax.ShapeDtypeStruct((nh, L // _KB, L, _KB), jnp.float32),
    )(q4, k4)


_W = 128
_NWIN = 320


def _sc_gather_m(table, addr, n_rows, u, l_k):
    n_sub, n_win, w = addr.shape
    rows_per = n_rows // n_sub
    n_grp = rows_per // 16
    assert n_grp * u * 16 == n_win * w
    mesh = plsc.VectorSubcoreMesh(
        core_axis_name="core", subcore_axis_name="subcore")

    @functools.partial(
        pl.kernel, mesh=mesh,
        out_type=jax.ShapeDtypeStruct((n_rows,), jnp.float32),
        scratch_types=[
            pltpu.VMEM((n_win, w), jnp.int32),
            pltpu.VMEM((n_win, w), jnp.float32),
            pltpu.VMEM((rows_per,), jnp.float32),
            pltpu.SemaphoreType.DMA,
            pltpu.SemaphoreType.DMA,
        ])
    def gather_kernel(x_hbm, i_hbm, o_hbm, idx_v, val_v, m_v, sem_i, sem_g):
        wid = lax.axis_index("core") * 16 + lax.axis_index("subcore")
        pltpu.async_copy(i_hbm.at[wid], idx_v, sem_i).wait()

        @pl.loop(0, n_win)
        def _fire(win):
            pltpu.async_copy(x_hbm.at[idx_v.at[win]], val_v.at[win], sem_g)

        @pl.loop(0, n_win)
        def _drain(win):
            pltpu.make_async_copy(
                x_hbm.at[idx_v.at[win]], val_v.at[win], sem_g).wait()

        for g in range(n_grp):
            base = g * 16 * u
            r0, c0 = base // w, base % w
            mx = val_v[r0, pl.ds(c0, 16)]
            sm = mx
            for s in range(1, u):
                off = base + s * 16
                v = val_v[off // w, pl.ds(off % w, 16)]
                mx = jnp.maximum(mx, v)
                sm = sm + v
            m_v[pl.ds(g * 16, 16)] = mx - sm * (1.0 / l_k)

        pltpu.sync_copy(m_v, o_hbm.at[pl.ds(wid * rows_per, rows_per)])

    return gather_kernel(table, addr)


def _sc_gather_rows(tables, addr, d):
    n_sub, n_tab, w = addr.shape
    mesh = plsc.VectorSubcoreMesh(
        core_axis_name="core", subcore_axis_name="subcore")

    @functools.partial(
        pl.kernel, mesh=mesh,
        out_type=jax.ShapeDtypeStruct((n_sub * n_tab * w, d), jnp.float32),
        scratch_types=[
            pltpu.VMEM((n_tab, w), jnp.int32),
            pltpu.VMEM((n_tab * w, d), jnp.float32),
            pltpu.SemaphoreType.DMA,
            pltpu.SemaphoreType.DMA,
        ])
    def rows_kernel(*refs):
        tabs = refs[:n_tab]
        i_hbm, o_hbm, idx_v, val_v, sem_i, sem_g = refs[n_tab:]
        wid = lax.axis_index("core") * 16 + lax.axis_index("subcore")
        pltpu.async_copy(i_hbm.at[wid], idx_v, sem_i).wait()
        for g in range(n_tab):
            pltpu.async_copy(
                tabs[g].at[idx_v.at[g]], val_v.at[pl.ds(g * w, w)], sem_g)
        for g in range(n_tab):
            pltpu.make_async_copy(
                tabs[g].at[idx_v.at[g]], val_v.at[pl.ds(g * w, w)],
                sem_g).wait()
        for g in range(n_tab):
            pltpu.sync_copy(
                val_v.at[pl.ds(g * w, w)],
                o_hbm.at[pl.ds(g * (n_sub * w) + wid * w, w)])

    return rows_kernel(*tables, addr)


def _topk_body(m_ref, idx_ref, *, u):
    m = m_ref[...]
    H, LB, C = m.shape
    gidx = (lax.broadcasted_iota(jnp.int32, (H, LB, C), 1) * C
            + lax.broadcasted_iota(jnp.int32, (H, LB, C), 2)
            ).astype(jnp.float32)
    neg = jnp.float32(-3.0e38)
    big = jnp.float32(2.0**30)
    for i in range(u):
        rm = jnp.max(jnp.max(m, axis=2, keepdims=True), axis=1,
                     keepdims=True)
        cand = jnp.where(m >= rm, gidx, big)
        pos = jnp.min(jnp.min(cand, axis=2, keepdims=True), axis=1,
                      keepdims=True)
        idx_ref[:, :, pl.ds(i, 1)] = pos.astype(jnp.int32)
        m = jnp.where(gidx == pos, neg, m)


def _topk(m3, u):
    H, LB, C = m3.shape
    return pl.pallas_call(
        functools.partial(_topk_body, u=u),
        grid=(1,),
        in_specs=[pl.BlockSpec((H, LB, C), lambda i: (0, 0, 0))],
        out_specs=pl.BlockSpec((H, 1, u), lambda i: (0, 0, 0)),
        out_shape=jax.ShapeDtypeStruct((H, 1, u), jnp.int32),
    )(m3)


_CB = 128


def _ctx_body(mtc_ref, mtr_ref, st_ref, v_ref, o_ref, *, scale):
    mt_col = mtc_ref[0]
    mt_row = mtr_ref[0]
    v = v_ref[0, 0]
    L, D = v.shape
    U = mt_col.shape[0]
    f32 = jnp.float32

    oh_lu = (lax.broadcasted_iota(jnp.int32, (L, U), 0) == mt_row)

    st = jnp.reshape(st_ref[...], (U, L)) * f32(scale)

    key_iota = lax.broadcasted_iota(jnp.int32, (U, L), 1)
    st = jnp.where(key_iota > mt_col, -jnp.inf, st)

    smax = jnp.max(st, axis=1, keepdims=True)
    e = jnp.exp(st - smax)
    attn = e / jnp.sum(e, axis=1, keepdims=True)

    upd = lax.dot_general(
        attn, v, (((1,), (0,)), ((), ())),
        preferred_element_type=f32)

    scat = lax.dot_general(
        oh_lu.astype(f32), upd, (((1,), (0,)), ((), ())),
        preferred_element_type=f32)

    selrow = jnp.sum(oh_lu.astype(f32), axis=1, keepdims=True) > 0

    tri = (lax.broadcasted_iota(jnp.int32, (_CB, _CB), 0)
           >= lax.broadcasted_iota(jnp.int32, (_CB, _CB), 1)).astype(f32)
    carry = jnp.zeros((1, D), f32)
    for b in range(L // _CB):
        lo = b * _CB
        blk = v[lo:lo + _CB, :]
        c = lax.dot_general(
            tri, blk, (((1,), (0,)), ((), ())),
            preferred_element_type=f32,
            precision=lax.Precision.HIGHEST) + carry
        o_ref[0, 0, lo:lo + _CB, :] = jnp.where(
            selrow[lo:lo + _CB, :], scat[lo:lo + _CB, :], c)
        carry = carry + jnp.sum(blk, axis=0, keepdims=True)


def _ctx(mt_col3, mt_row3, st_all, v4, scale):
    _, H, L, D = v4.shape
    U = mt_col3.shape[1]
    rows_h = U * L // D
    return pl.pallas_call(
        functools.partial(_ctx_body, scale=scale),
        grid=(H,),
        in_specs=[
            pl.BlockSpec((1, U, 1), lambda h: (h, 0, 0)),
            pl.BlockSpec((1, 1, U), lambda h: (h, 0, 0)),
            pl.BlockSpec((rows_h, D), lambda h: (h, 0)),
            pl.BlockSpec((1, 1, L, D), lambda h: (0, h, 0, 0)),
        ],
        out_specs=pl.BlockSpec((1, 1, L, D), lambda h: (0, h, 0, 0)),
        out_shape=jax.ShapeDtypeStruct((1, H, L, D), jnp.float32),
    )(mt_col3, mt_row3, st_all, v4)


_NCHUNK = 4


def kernel(queries, keys, values, attn_mask):
    B, H, L_Q, E = queries.shape
    L_K = keys.shape[2]
    factor = 5
    scale = 1.0 / math.sqrt(E)
    u_part = min(factor * math.ceil(math.log(L_K)), L_K)
    u = min(factor * math.ceil(math.log(L_Q)), L_Q)
    hc = H // _NCHUNK

    with jax.ensure_compile_time_eval():
        skey = jax.random.key(12345)
        idx = jax.random.randint(skey, (L_Q, u_part), 0, L_K)
        hb = jnp.arange(hc, dtype=jnp.int32)[:, None, None]
        lb = jnp.arange(L_Q, dtype=jnp.int32)[None, :, None]
        jb = (idx // _KB).astype(jnp.int32)[None]
        cb = (idx % _KB).astype(jnp.int32)[None]
        addr_c = ((hb * (L_K // _KB) + jb) * (L_Q * _KB)
                  + lb * _KB + cb)
        addr_c = (addr_c.reshape(32, hc * L_Q // (32 * 16), 16, u_part)
                  .transpose(0, 1, 3, 2)
                  .reshape(32, hc * u_part * L_Q // (32 * _W), _W))

    m_parts, s_tables = [], []
    for g in range(_NCHUNK):
        s_g = _compute_scores(queries, keys, g * hc, hc)
        s_tables.append(s_g.reshape(hc * (L_K // _KB) * L_Q, _KB))
        m_g = _sc_gather_m(s_g.reshape(hc * L_Q * L_K), addr_c,
                           hc * L_Q, u_part, L_K)
        m_parts.append(m_g.reshape(hc, L_Q // 128, 128))

    m3 = jnp.concatenate(m_parts, axis=0)
    mtop = _topk(m3, u)

    with jax.ensure_compile_time_eval():
        base = ((jnp.arange(hc, dtype=jnp.int32)[:, None, None] * (L_K // _KB)
                 + jnp.arange(L_K // _KB, dtype=jnp.int32)[None, None, :])
                * L_Q)
    m2 = mtop.reshape(H, u)
    addr_st = jnp.stack(
        [(base + m2[g * hc:(g + 1) * hc, :, None]).reshape(32, -1)
         for g in range(_NCHUNK)], axis=1)
    st_all = _sc_gather_rows(s_tables, addr_st, _KB)

    return _ctx(mtop.reshape(H, u, 1), mtop.reshape(H, 1, u),
                st_all, values, scale)
